# baseline (speedup 1.0000x reference)
import functools

import jax
import jax.numpy as jnp
from jax import lax
from jax.experimental import pallas as pl
from jax.experimental.pallas import tpu as pltpu
from jax.experimental.pallas import tpu_sc as plsc

BS = 8
NP = 16384
CIN = 128
COUT = 128

SC_ROWS = 2048
NQ = 4
QROWS = SC_ROWS // NQ
CH = 256
NCHUNK = QROWS // CH

QG = NP // NQ

BLK = 1024
NB = (NP - SC_ROWS) // BLK


def _sc_reduce_body(mad_ref, adj_ref, pt_ref, cst_ref,
                    gmad_ref, gadj_ref, gpt_ref, gxyz_ref,
                    out_ref, outg_ref, buf, gbuf, accbuf, sem0, sem1):
    c = lax.axis_index("c")
    s = lax.axis_index("s")
    wid = s * 2 + c
    q = wid // BS
    b = wid % BS
    base = q * QROWS
    sems = (sem0, sem1)
    for h, ref in enumerate((mad_ref, adj_ref, pt_ref, cst_ref)):
        pltpu.async_copy(ref.at[b, pl.ds(base, CH), :], buf.at[0], sem0)
        pltpu.async_copy(ref.at[b, pl.ds(base + CH, CH), :], buf.at[1], sem1)
        accs = tuple(jnp.full((16,), -jnp.inf, jnp.float32) for _ in range(8))

        def pair(i2, accs, _ref=ref):
            for sl in range(2):
                pltpu.make_async_copy(
                    _ref.at[b, pl.ds(base, CH), :], buf.at[sl],
                    sems[sl]).wait()

                def row(r, a, _sl=sl):
                    return tuple(
                        jnp.maximum(a[v], buf[_sl, r, pl.ds(v * 16, 16)])
                        for v in range(8))

                accs = lax.fori_loop(0, CH, row, accs, unroll=4)
                nxt = i2 * 2 + sl + 2

                @pl.when(nxt < NCHUNK)
                def _():
                    pltpu.async_copy(
                        _ref.at[b, pl.ds(base + nxt * CH, CH), :],
                        buf.at[sl], sems[sl])
            return accs

        accs = lax.fori_loop(0, NCHUNK // 2, pair, accs)
        for v in range(8):
            accbuf[pl.ds(v * 16, 16)] = accs[v]
        pltpu.sync_copy(accbuf, out_ref.at[h, q, b])

    out_accs = []
    for gref, k in ((gmad_ref, 3), (gadj_ref, 2), (gpt_ref, 4),
                    (gxyz_ref, 3)):
        n = QG * k
        pltpu.sync_copy(gref.at[b, pl.ds(q * n, n)], gbuf.at[pl.ds(0, n)])
        if k == 3:
            ga = tuple(
                jnp.full((16,), -jnp.inf, jnp.float32) for _ in range(3))

            def g3(i, a):
                return tuple(
                    jnp.maximum(a[p], gbuf[pl.ds(i * 48 + 16 * p, 16)])
                    for p in range(3))

            out_accs += list(lax.fori_loop(0, n // 48, g3, ga, unroll=4))
        else:
            ga = jnp.full((16,), -jnp.inf, jnp.float32)

            def g1(i, a):
                return jnp.maximum(a, gbuf[pl.ds(i * 16, 16)])

            out_accs.append(lax.fori_loop(0, n // 16, g1, ga, unroll=8))
    for v, a in enumerate(out_accs):
        accbuf[pl.ds(v * 16, 16)] = a
    pltpu.sync_copy(accbuf, outg_ref.at[wid])


def _tc_reduce_body(mad_f, adj_f, pt_f, cst_f,
                    o_mad, o_adj, o_pt, o_cst,
                    a_mad, a_adj, a_pt, a_cst):
    j = pl.program_id(0)

    fm = jnp.max(mad_f[...], axis=1)
    fa = jnp.max(adj_f[...], axis=1)
    fp = jnp.max(pt_f[...], axis=1)
    fc = jnp.max(cst_f[...], axis=1)

    @pl.when(j == 0)
    def _():
        a_mad[...] = fm
        a_adj[...] = fa
        a_pt[...] = fp
        a_cst[...] = fc

    @pl.when(j > 0)
    def _():
        a_mad[...] = jnp.maximum(a_mad[...], fm)
        a_adj[...] = jnp.maximum(a_adj[...], fa)
        a_pt[...] = jnp.maximum(a_pt[...], fp)
        a_cst[...] = jnp.maximum(a_cst[...], fc)

    @pl.when(j == NB - 1)
    def _():
        o_mad[...] = a_mad[...]
        o_adj[...] = a_adj[...]
        o_pt[...] = a_pt[...]
        o_cst[...] = a_cst[...]


def _head(acc_fea, acc_geom, k, reps, W1, b1, W2, b2, out_ref):
    h = jax.lax.dot(acc_fea, W1[0:CIN, :], preferred_element_type=jnp.float32)
    tail = W1[CIN:, :]
    for r in range(k * reps):
        h = h + acc_geom[:, (r % k):(r % k) + 1] * tail[r:r + 1, :]
    h = jax.nn.relu(h + b1[...])
    h = jax.nn.relu(
        jax.lax.dot(h, W2[...], preferred_element_type=jnp.float32) + b2[...])
    out_ref[...] = h


def _finish_body(p_ref, pg_ref, t_mad, t_adj, t_pt, t_cst,
                 Wm1, bm1, Wm2, bm2, Wa1, ba1, Wa2, ba2,
                 Wp1, bp1, Wp2, bp2, Wc1, bc1, Wc2, bc2,
                 o_mad, o_adj, o_pt, o_cst):
    a_mad = jnp.maximum(jnp.max(p_ref[0], axis=0), t_mad[...])
    a_adj = jnp.maximum(jnp.max(p_ref[1], axis=0), t_adj[...])
    a_pt = jnp.maximum(jnp.max(p_ref[2], axis=0), t_pt[...])
    a_cst = jnp.maximum(jnp.max(p_ref[3], axis=0), t_cst[...])

    G = jnp.max(pg_ref[...], axis=0)
    lane = jax.lax.broadcasted_iota(jnp.int32, (BS, 16), 1)

    def fold3(s0):
        cols = []
        for col in range(3):
            m = None
            for p in range(3):
                msk = ((16 * p + lane) % 3) == col
                v = jnp.where(msk, G[:, s0 + p, :], -jnp.inf)
                r = jnp.max(v, axis=1, keepdims=True)
                m = r if m is None else jnp.maximum(m, r)
            cols.append(m)
        return jnp.concatenate(cols, axis=1)

    def fold1(s0, k):
        cols = []
        for col in range(k):
            v = jnp.where((lane % k) == col, G[:, s0, :], -jnp.inf)
            cols.append(jnp.max(v, axis=1, keepdims=True))
        return jnp.concatenate(cols, axis=1)

    g_mad = fold3(0)
    g_adj = fold1(3, 2)
    g_pt = fold1(4, 4)
    g_xyz = fold3(5)

    _head(a_mad, g_mad, 3, 1, Wm1, bm1, Wm2, bm2, o_mad)
    _head(a_adj, g_adj, 2, 2, Wa1, ba1, Wa2, ba2, o_adj)
    _head(a_pt, g_pt, 4, 2, Wp1, bp1, Wp2, bp2, o_pt)
    _head(a_cst, g_xyz, 3, 1, Wc1, bc1, Wc2, bc2, o_cst)


def kernel(xyz, mad, adj, pt, mad_fea, adj_fea, pt_fea, cst_fea,
           W_mad1, b_mad1, W_mad2, b_mad2,
           W_adj1, b_adj1, W_adj2, b_adj2,
           W_pt1, b_pt1, W_pt2, b_pt2,
           W_cst1, b_cst1, W_cst2, b_cst2):
    sc_reduce = pl.kernel(
        _sc_reduce_body,
        out_type=[
            jax.ShapeDtypeStruct((4, NQ, BS, CIN), jnp.float32),
            jax.ShapeDtypeStruct((NQ * BS, 128), jnp.float32),
        ],
        mesh=plsc.VectorSubcoreMesh(core_axis_name="c", subcore_axis_name="s"),
        scratch_types=[
            pltpu.VMEM((2, CH, CIN), jnp.float32),
            pltpu.VMEM((QG * 4,), jnp.float32),
            pltpu.VMEM((CIN,), jnp.float32),
            pltpu.SemaphoreType.DMA,
            pltpu.SemaphoreType.DMA,
        ],
    )
    sc_partials, sc_geom = sc_reduce(
        mad_fea, adj_fea, pt_fea, cst_fea,
        mad.reshape(BS, NP * 3), adj.reshape(BS, NP * 2),
        pt.reshape(BS, NP * 4), xyz.reshape(BS, NP * 3))

    fea_spec = pl.BlockSpec((BS, BLK, CIN),
                            lambda j: (0, SC_ROWS // BLK + j, 0))
    fea_out_spec = pl.BlockSpec((BS, CIN), lambda j: (0, 0))

    tc_parts = pl.pallas_call(
        _tc_reduce_body,
        grid=(NB,),
        in_specs=[fea_spec] * 4,
        out_specs=[fea_out_spec] * 4,
        out_shape=[jax.ShapeDtypeStruct((BS, CIN), jnp.float32)] * 4,
        scratch_shapes=[pltpu.VMEM((BS, CIN), jnp.float32)] * 4,
        compiler_params=pltpu.CompilerParams(
            dimension_semantics=("arbitrary",)),
    )(mad_fea, adj_fea, pt_fea, cst_fea)

    b_mad1, b_mad2 = b_mad1.reshape(1, -1), b_mad2.reshape(1, -1)
    b_adj1, b_adj2 = b_adj1.reshape(1, -1), b_adj2.reshape(1, -1)
    b_pt1, b_pt2 = b_pt1.reshape(1, -1), b_pt2.reshape(1, -1)
    b_cst1, b_cst2 = b_cst1.reshape(1, -1), b_cst2.reshape(1, -1)
    weights = (W_mad1, b_mad1, W_mad2, b_mad2,
               W_adj1, b_adj1, W_adj2, b_adj2,
               W_pt1, b_pt1, W_pt2, b_pt2,
               W_cst1, b_cst1, W_cst2, b_cst2)

    res = pl.pallas_call(
        _finish_body,
        out_shape=[jax.ShapeDtypeStruct((BS, COUT), jnp.float32)] * 4,
    )(sc_partials, sc_geom.reshape(NQ, BS, 8, 16), *tc_parts, *weights)
    return tuple(res)


# speedup vs baseline: 2.9144x; 2.9144x over previous
import functools

import jax
import jax.numpy as jnp
from jax import lax
from jax.experimental import pallas as pl
from jax.experimental.pallas import tpu as pltpu
from jax.experimental.pallas import tpu_sc as plsc

BS = 8
NP = 16384
CIN = 128
COUT = 128

SC_ROWS = 2048
NQ = 4
QROWS = SC_ROWS // NQ
CH = 256
NCHUNK = QROWS // CH

BLK = 512
NB = (NP - SC_ROWS) // BLK


def _sc_reduce_body(mad_ref, adj_ref, pt_ref, cst_ref, out_ref,
                    buf, accbuf, sem0, sem1):
    c = lax.axis_index("c")
    s = lax.axis_index("s")
    wid = s * 2 + c
    q = wid // BS
    b = wid % BS
    base = q * QROWS
    sems = (sem0, sem1)
    for h, ref in enumerate((mad_ref, adj_ref, pt_ref, cst_ref)):
        pltpu.async_copy(ref.at[b, pl.ds(base, CH), :], buf.at[0], sem0)
        pltpu.async_copy(ref.at[b, pl.ds(base + CH, CH), :], buf.at[1], sem1)
        accs = tuple(jnp.full((16,), -jnp.inf, jnp.float32) for _ in range(8))

        def pair(i2, accs, _ref=ref):
            for sl in range(2):
                pltpu.make_async_copy(
                    _ref.at[b, pl.ds(base, CH), :], buf.at[sl],
                    sems[sl]).wait()

                def row(r, a, _sl=sl):
                    return tuple(
                        jnp.maximum(a[v], buf[_sl, r, pl.ds(v * 16, 16)])
                        for v in range(8))

                accs = lax.fori_loop(0, CH, row, accs, unroll=4)
                nxt = i2 * 2 + sl + 2

                @pl.when(nxt < NCHUNK)
                def _():
                    pltpu.async_copy(
                        _ref.at[b, pl.ds(base + nxt * CH, CH), :],
                        buf.at[sl], sems[sl])
            return accs

        accs = lax.fori_loop(0, NCHUNK // 2, pair, accs)
        for v in range(8):
            accbuf[pl.ds(v * 16, 16)] = accs[v]
        pltpu.sync_copy(accbuf, out_ref.at[h, q, b])


def _tc_reduce_body(mad_f, adj_f, pt_f, cst_f, madg, adjg, ptg, xyzg,
                    o_mad, o_adj, o_pt, o_cst, og_mad, og_adj, og_pt, og_cst,
                    a_mad, a_adj, a_pt, a_cst):
    j = pl.program_id(0)

    fm = jnp.max(mad_f[...], axis=1)
    fa = jnp.max(adj_f[...], axis=1)
    fp = jnp.max(pt_f[...], axis=1)
    fc = jnp.max(cst_f[...], axis=1)

    @pl.when(j == 0)
    def _():
        a_mad[...] = fm
        a_adj[...] = fa
        a_pt[...] = fp
        a_cst[...] = fc

    @pl.when(j > 0)
    def _():
        a_mad[...] = jnp.maximum(a_mad[...], fm)
        a_adj[...] = jnp.maximum(a_adj[...], fa)
        a_pt[...] = jnp.maximum(a_pt[...], fp)
        a_cst[...] = jnp.maximum(a_cst[...], fc)

    @pl.when(j == NB - 1)
    def _():
        o_mad[...] = a_mad[...]
        o_adj[...] = a_adj[...]
        o_pt[...] = a_pt[...]
        o_cst[...] = a_cst[...]
        og_mad[...] = jnp.max(madg[...], axis=2)
        og_adj[...] = jnp.max(adjg[...], axis=2)
        og_pt[...] = jnp.max(ptg[...], axis=2)
        og_cst[...] = jnp.max(xyzg[...], axis=2)


def _head(acc_fea, acc_geom, k, reps, W1, b1, W2, b2, out_ref):
    h = jax.lax.dot(acc_fea, W1[0:CIN, :], preferred_element_type=jnp.float32)
    tail = W1[CIN:, :]
    for r in range(k * reps):
        h = h + acc_geom[:, (r % k):(r % k) + 1] * tail[r:r + 1, :]
    h = jax.nn.relu(h + b1[...])
    h = jax.nn.relu(
        jax.lax.dot(h, W2[...], preferred_element_type=jnp.float32) + b2[...])
    out_ref[...] = h


def _finish_body(p_ref, t_mad, t_adj, t_pt, t_cst,
                 g_mad, g_adj, g_pt, g_cst,
                 Wm1, bm1, Wm2, bm2, Wa1, ba1, Wa2, ba2,
                 Wp1, bp1, Wp2, bp2, Wc1, bc1, Wc2, bc2,
                 o_mad, o_adj, o_pt, o_cst):
    a_mad = jnp.maximum(jnp.max(p_ref[0], axis=0), t_mad[...])
    a_adj = jnp.maximum(jnp.max(p_ref[1], axis=0), t_adj[...])
    a_pt = jnp.maximum(jnp.max(p_ref[2], axis=0), t_pt[...])
    a_cst = jnp.maximum(jnp.max(p_ref[3], axis=0), t_cst[...])
    _head(a_mad, g_mad[...], 3, 1, Wm1, bm1, Wm2, bm2, o_mad)
    _head(a_adj, g_adj[...], 2, 2, Wa1, ba1, Wa2, ba2, o_adj)
    _head(a_pt, g_pt[...], 4, 2, Wp1, bp1, Wp2, bp2, o_pt)
    _head(a_cst, g_cst[...], 3, 1, Wc1, bc1, Wc2, bc2, o_cst)


def kernel(xyz, mad, adj, pt, mad_fea, adj_fea, pt_fea, cst_fea,
           W_mad1, b_mad1, W_mad2, b_mad2,
           W_adj1, b_adj1, W_adj2, b_adj2,
           W_pt1, b_pt1, W_pt2, b_pt2,
           W_cst1, b_cst1, W_cst2, b_cst2):
    sc_reduce = pl.kernel(
        _sc_reduce_body,
        out_type=jax.ShapeDtypeStruct((4, NQ, BS, CIN), jnp.float32),
        mesh=plsc.VectorSubcoreMesh(core_axis_name="c", subcore_axis_name="s"),
        scratch_types=[
            pltpu.VMEM((2, CH, CIN), jnp.float32),
            pltpu.VMEM((CIN,), jnp.float32),
            pltpu.SemaphoreType.DMA,
            pltpu.SemaphoreType.DMA,
        ],
    )
    sc_partials = sc_reduce(mad_fea, adj_fea, pt_fea, cst_fea)

    fea_spec = pl.BlockSpec((BS, BLK, CIN),
                            lambda j: (0, SC_ROWS // BLK + j, 0))

    def geo_spec(k):
        return pl.BlockSpec((BS, k, NP), lambda j: (0, 0, 0))

    fea_out_spec = pl.BlockSpec((BS, CIN), lambda j: (0, 0))

    def geo_out_spec(k):
        return pl.BlockSpec((BS, k), lambda j: (0, 0))

    tc_parts = pl.pallas_call(
        _tc_reduce_body,
        grid=(NB,),
        in_specs=[fea_spec] * 4
        + [geo_spec(3), geo_spec(2), geo_spec(4), geo_spec(3)],
        out_specs=[fea_out_spec] * 4
        + [geo_out_spec(3), geo_out_spec(2), geo_out_spec(4), geo_out_spec(3)],
        out_shape=[jax.ShapeDtypeStruct((BS, CIN), jnp.float32)] * 4
        + [jax.ShapeDtypeStruct((BS, 3), jnp.float32),
           jax.ShapeDtypeStruct((BS, 2), jnp.float32),
           jax.ShapeDtypeStruct((BS, 4), jnp.float32),
           jax.ShapeDtypeStruct((BS, 3), jnp.float32)],
        scratch_shapes=[pltpu.VMEM((BS, CIN), jnp.float32)] * 4,
        compiler_params=pltpu.CompilerParams(
            dimension_semantics=("arbitrary",)),
    )(mad_fea, adj_fea, pt_fea, cst_fea,
      mad.transpose(0, 2, 1), adj.transpose(0, 2, 1),
      pt.transpose(0, 2, 1), xyz.transpose(0, 2, 1))

    b_mad1, b_mad2 = b_mad1.reshape(1, -1), b_mad2.reshape(1, -1)
    b_adj1, b_adj2 = b_adj1.reshape(1, -1), b_adj2.reshape(1, -1)
    b_pt1, b_pt2 = b_pt1.reshape(1, -1), b_pt2.reshape(1, -1)
    b_cst1, b_cst2 = b_cst1.reshape(1, -1), b_cst2.reshape(1, -1)
    weights = (W_mad1, b_mad1, W_mad2, b_mad2,
               W_adj1, b_adj1, W_adj2, b_adj2,
               W_pt1, b_pt1, W_pt2, b_pt2,
               W_cst1, b_cst1, W_cst2, b_cst2)

    res = pl.pallas_call(
        _finish_body,
        out_shape=[jax.ShapeDtypeStruct((BS, COUT), jnp.float32)] * 4,
    )(sc_partials, *tc_parts, *weights)
    return tuple(res)


# speedup vs baseline: 3.0668x; 1.0523x over previous
import jax
import jax.numpy as jnp
from jax import lax
from jax.experimental import pallas as pl
from jax.experimental.pallas import tpu as pltpu
from jax.experimental.pallas import tpu_sc as plsc

BS = 8
NP = 16384
CIN = 128
COUT = 128

SC_ROWS = 2048
NQ = 4
QROWS = SC_ROWS // NQ
CH = 256
NCHUNK = QROWS // CH

BLK = 512
NB = (NP - SC_ROWS) // BLK


def _sc_reduce_body(mad_ref, adj_ref, pt_ref, cst_ref, out_ref,
                    buf, accbuf, sem0, sem1):
    c = lax.axis_index("c")
    s = lax.axis_index("s")
    wid = s * 2 + c
    q = wid // BS
    b = wid % BS
    base = q * QROWS
    sems = (sem0, sem1)
    for h, ref in enumerate((mad_ref, adj_ref, pt_ref, cst_ref)):
        pltpu.async_copy(ref.at[b, pl.ds(base, CH), :], buf.at[0], sem0)
        pltpu.async_copy(ref.at[b, pl.ds(base + CH, CH), :], buf.at[1], sem1)
        accs = tuple(jnp.full((16,), -jnp.inf, jnp.float32) for _ in range(8))

        def pair(i2, accs, _ref=ref):
            for sl in range(2):
                pltpu.make_async_copy(
                    _ref.at[b, pl.ds(base, CH), :], buf.at[sl],
                    sems[sl]).wait()

                def row(r, a, _sl=sl):
                    return tuple(
                        jnp.maximum(a[v], buf[_sl, r, pl.ds(v * 16, 16)])
                        for v in range(8))

                accs = lax.fori_loop(0, CH, row, accs, unroll=4)
                nxt = i2 * 2 + sl + 2

                @pl.when(nxt < NCHUNK)
                def _():
                    pltpu.async_copy(
                        _ref.at[b, pl.ds(base + nxt * CH, CH), :],
                        buf.at[sl], sems[sl])
            return accs

        accs = lax.fori_loop(0, NCHUNK // 2, pair, accs)
        for v in range(8):
            accbuf[pl.ds(v * 16, 16)] = accs[v]
        pltpu.sync_copy(accbuf, out_ref.at[h, q, b])


def _tc_reduce_body(mad_f, adj_f, pt_f, cst_f, madg, adjg, ptg, xyzg,
                    o_mad, o_adj, o_pt, o_cst, og_mad, og_adj, og_pt, og_cst,
                    a_mad, a_adj, a_pt, a_cst):
    j = pl.program_id(0)

    fm = jnp.max(mad_f[...], axis=1)
    fa = jnp.max(adj_f[...], axis=1)
    fp = jnp.max(pt_f[...], axis=1)
    fc = jnp.max(cst_f[...], axis=1)

    @pl.when(j == 0)
    def _():
        a_mad[...] = fm
        a_adj[...] = fa
        a_pt[...] = fp
        a_cst[...] = fc

    @pl.when(j > 0)
    def _():
        a_mad[...] = jnp.maximum(a_mad[...], fm)
        a_adj[...] = jnp.maximum(a_adj[...], fa)
        a_pt[...] = jnp.maximum(a_pt[...], fp)
        a_cst[...] = jnp.maximum(a_cst[...], fc)

    @pl.when(j == NB - 1)
    def _():
        o_mad[...] = a_mad[...]
        o_adj[...] = a_adj[...]
        o_pt[...] = a_pt[...]
        o_cst[...] = a_cst[...]
        og_mad[...] = jnp.max(madg[...], axis=2)
        og_adj[...] = jnp.max(adjg[...], axis=2)
        og_pt[...] = jnp.max(ptg[...], axis=2)
        og_cst[...] = jnp.max(xyzg[...], axis=2)


def _head(acc_fea, acc_geom, k, reps, W1, b1, W2, b2, out_ref):
    h = jax.lax.dot(acc_fea, W1[0:CIN, :], preferred_element_type=jnp.float32)
    tail = W1[CIN:, :]
    for r in range(k * reps):
        h = h + acc_geom[:, (r % k):(r % k) + 1] * tail[r:r + 1, :]
    h = jax.nn.relu(h + b1[...])
    h = jax.nn.relu(
        jax.lax.dot(h, W2[...], preferred_element_type=jnp.float32) + b2[...])
    out_ref[...] = h


def _finish_body(p_ref, t_mad, t_adj, t_pt, t_cst,
                 g_mad, g_adj, g_pt, g_cst,
                 Wm1, bm1, Wm2, bm2, Wa1, ba1, Wa2, ba2,
                 Wp1, bp1, Wp2, bp2, Wc1, bc1, Wc2, bc2,
                 o_mad, o_adj, o_pt, o_cst):
    a_mad = jnp.maximum(jnp.max(p_ref[0], axis=0), t_mad[...])
    a_adj = jnp.maximum(jnp.max(p_ref[1], axis=0), t_adj[...])
    a_pt = jnp.maximum(jnp.max(p_ref[2], axis=0), t_pt[...])
    a_cst = jnp.maximum(jnp.max(p_ref[3], axis=0), t_cst[...])
    _head(a_mad, g_mad[...], 3, 1, Wm1, bm1, Wm2, bm2, o_mad)
    _head(a_adj, g_adj[...], 2, 2, Wa1, ba1, Wa2, ba2, o_adj)
    _head(a_pt, g_pt[...], 4, 2, Wp1, bp1, Wp2, bp2, o_pt)
    _head(a_cst, g_cst[...], 3, 1, Wc1, bc1, Wc2, bc2, o_cst)


def kernel(xyz, mad, adj, pt, mad_fea, adj_fea, pt_fea, cst_fea,
           W_mad1, b_mad1, W_mad2, b_mad2,
           W_adj1, b_adj1, W_adj2, b_adj2,
           W_pt1, b_pt1, W_pt2, b_pt2,
           W_cst1, b_cst1, W_cst2, b_cst2):
    sc_reduce = pl.kernel(
        _sc_reduce_body,
        out_type=jax.ShapeDtypeStruct((4, NQ, BS, CIN), jnp.float32),
        mesh=plsc.VectorSubcoreMesh(core_axis_name="c", subcore_axis_name="s"),
        scratch_types=[
            pltpu.VMEM((2, CH, CIN), jnp.float32),
            pltpu.VMEM((CIN,), jnp.float32),
            pltpu.SemaphoreType.DMA,
            pltpu.SemaphoreType.DMA,
        ],
    )
    sc_partials = sc_reduce(mad_fea, adj_fea, pt_fea, cst_fea)

    fea_spec = pl.BlockSpec((BS, BLK, CIN),
                            lambda j: (0, SC_ROWS // BLK + j, 0))

    def geo_spec(k):
        return pl.BlockSpec((BS, k, NP), lambda j: (0, 0, 0))

    fea_out_spec = pl.BlockSpec((BS, CIN), lambda j: (0, 0))

    def geo_out_spec(k):
        return pl.BlockSpec((BS, k), lambda j: (0, 0))

    tc_parts = pl.pallas_call(
        _tc_reduce_body,
        grid=(NB,),
        in_specs=[fea_spec] * 4
        + [geo_spec(3), geo_spec(2), geo_spec(4), geo_spec(3)],
        out_specs=[fea_out_spec] * 4
        + [geo_out_spec(3), geo_out_spec(2), geo_out_spec(4), geo_out_spec(3)],
        out_shape=[jax.ShapeDtypeStruct((BS, CIN), jnp.float32)] * 4
        + [jax.ShapeDtypeStruct((BS, 3), jnp.float32),
           jax.ShapeDtypeStruct((BS, 2), jnp.float32),
           jax.ShapeDtypeStruct((BS, 4), jnp.float32),
           jax.ShapeDtypeStruct((BS, 3), jnp.float32)],
        scratch_shapes=[pltpu.VMEM((BS, CIN), jnp.float32)] * 4,
        compiler_params=pltpu.CompilerParams(
            dimension_semantics=("arbitrary",)),
    )(mad_fea, adj_fea, pt_fea, cst_fea,
      mad.transpose(0, 2, 1), adj.transpose(0, 2, 1),
      pt.transpose(0, 2, 1), xyz.transpose(0, 2, 1))

    b_mad1, b_mad2 = b_mad1.reshape(1, -1), b_mad2.reshape(1, -1)
    b_adj1, b_adj2 = b_adj1.reshape(1, -1), b_adj2.reshape(1, -1)
    b_pt1, b_pt2 = b_pt1.reshape(1, -1), b_pt2.reshape(1, -1)
    b_cst1, b_cst2 = b_cst1.reshape(1, -1), b_cst2.reshape(1, -1)
    weights = (W_mad1, b_mad1, W_mad2, b_mad2,
               W_adj1, b_adj1, W_adj2, b_adj2,
               W_pt1, b_pt1, W_pt2, b_pt2,
               W_cst1, b_cst1, W_cst2, b_cst2)

    res = pl.pallas_call(
        _finish_body,
        out_shape=[jax.ShapeDtypeStruct((BS, COUT), jnp.float32)] * 4,
    )(sc_partials, *tc_parts, *weights)
    return tuple(res)
